# baseline (speedup 1.0000x reference)
import jax
import jax.numpy as jnp
from jax.experimental import pallas as pl


def _copy_kernel(in_ref, out_ref):
    out_ref[...] = in_ref[...]


def kernel(seq_len, position_embedding):
    del seq_len
    M, H = position_embedding.shape
    BM = 512
    return pl.pallas_call(
        _copy_kernel,
        grid=(M // BM,),
        in_specs=[pl.BlockSpec((BM, H), lambda i: (i, 0))],
        out_specs=pl.BlockSpec((BM, H), lambda i: (i, 0)),
        out_shape=jax.ShapeDtypeStruct((M, H), position_embedding.dtype),
    )(position_embedding)
